# Initial kernel scaffold; baseline (speedup 1.0000x reference)
import functools

import jax
import jax.numpy as jnp
from jax import lax
from jax.experimental import pallas as pl
from jax.experimental.pallas import tpu as pltpu
from jax.experimental.pallas import tpu_sc as plsc

B = 16384
D = 128
PREVIEW = 5
S = 7
NC = 2
NS = 16
L = 16
NW = NC * NS
BPW = B // NW
CB = 16
NCHUNK = BPW // CB
ROWS = CB * S
_ROLE_IDS = (0, 1, 2, 2, 2, 2, 2)


@functools.partial(
    pl.kernel,
    out_type=jax.ShapeDtypeStruct((B * S, D), jnp.float32),
    mesh=plsc.VectorSubcoreMesh(core_axis_name="c", subcore_axis_name="s"),
    scratch_types=dict(
        cur_v=pltpu.VMEM((BPW,), jnp.int32),
        hold_v=pltpu.VMEM((BPW,), jnp.int32),
        hh_v=pltpu.VMEM((BPW,), jnp.float32),
        pv_v=pltpu.VMEM((BPW * PREVIEW,), jnp.int32),
        pe_v=pltpu.VMEM((8, D), jnp.float32),
        role_v=pltpu.VMEM((3, D), jnp.float32),
        slot_v=pltpu.VMEM((S, D), jnp.float32),
        type_v=pltpu.VMEM((D,), jnp.float32),
        hhe_v=pltpu.VMEM((D,), jnp.float32),
        tbl_v=pltpu.VMEM((56, D), jnp.float32),
        shared_tbl=pltpu.VMEM_SHARED((56, D), jnp.float32),
        fidx_v=pltpu.VMEM((ROWS,), jnp.int32),
        rows_v=pltpu.VMEM((ROWS, D), jnp.float32),
        sem=pltpu.SemaphoreType.DMA,
    ),
)
def _pq_embed_sc(cur_hbm, hold_hbm, hh_hbm, pv_hbm, pe_hbm, role_hbm,
                 slot_hbm, hhe_hbm, type_hbm, out_hbm, *, cur_v, hold_v,
                 hh_v, pv_v, pe_v, role_v, slot_v, type_v, hhe_v, tbl_v,
                 shared_tbl, fidx_v, rows_v, sem):
    cid = lax.axis_index("c")
    sid = lax.axis_index("s")
    wid = sid * NC + cid
    base = wid * BPW

    pltpu.sync_copy(cur_hbm.at[pl.ds(base, BPW)], cur_v)
    pltpu.sync_copy(hold_hbm.at[pl.ds(base, BPW)], hold_v)
    pltpu.sync_copy(hh_hbm.at[pl.ds(base, BPW)], hh_v)
    pltpu.sync_copy(pv_hbm.at[pl.ds(base * PREVIEW, BPW * PREVIEW)], pv_v)
    pltpu.sync_copy(hhe_hbm, hhe_v)

    @pl.when(sid == 0)
    def _build_table():
        pltpu.sync_copy(pe_hbm, pe_v)
        pltpu.sync_copy(role_hbm, role_v)
        pltpu.sync_copy(slot_hbm, slot_v)
        pltpu.sync_copy(type_hbm, type_v)
        for d in range(D // L):
            dsl = pl.ds(d * L, L)
            t16 = type_v[dsl]
            for s in range(S):
                bias = role_v[_ROLE_IDS[s], dsl] + slot_v[s, dsl] + t16
                for p in range(8):
                    tbl_v[s * 8 + p, dsl] = pe_v[p, dsl] + bias
        pltpu.sync_copy(tbl_v, shared_tbl)

    plsc.subcore_barrier()

    iota = lax.iota(jnp.int32, L)
    hhe_g = [hhe_v[pl.ds(d * L, L)] for d in range(D // L)]

    def chunk_body(ci, carry):
        cb = ci * CB
        cur16 = cur_v[pl.ds(cb, L)]
        plsc.store_scatter(fidx_v, [iota * S], cur16)
        hold16 = hold_v[pl.ds(cb, L)]
        plsc.store_scatter(fidx_v, [iota * S + 1], hold16 + 8)
        for g in range(CB * PREVIEW // L):
            j = iota + g * L
            bb = j // PREVIEW
            ss = j - bb * PREVIEW
            pv16 = pv_v[pl.ds(cb * PREVIEW + g * L, L)]
            plsc.store_scatter(fidx_v, [bb * S + ss + 2],
                               pv16 + (ss + 2) * 8)
        pltpu.async_copy(shared_tbl.at[fidx_v], rows_v, sem).wait()
        for b2 in range(CB):
            hidx = jnp.full((L,), b2, jnp.int32) + cb
            splat = plsc.load_gather(hh_v, [hidx])
            for d in range(D // L):
                dsl = pl.ds(d * L, L)
                rows_v[b2 * S + 1, dsl] = (rows_v[b2 * S + 1, dsl]
                                           + splat * hhe_g[d])
        pltpu.sync_copy(rows_v, out_hbm.at[pl.ds((base + cb) * S, ROWS)])
        return carry

    lax.fori_loop(0, NCHUNK, chunk_body, 0)


def kernel(current, hold, has_held, preview, piece_embed, role_embed,
           slot_pos_embed, has_held_embed, type_embed):
    b = current.shape[0]
    out = _pq_embed_sc(
        current.astype(jnp.int32),
        hold.astype(jnp.int32),
        has_held.astype(jnp.float32),
        preview.astype(jnp.int32).reshape(b * PREVIEW),
        piece_embed.astype(jnp.float32),
        role_embed.astype(jnp.float32),
        slot_pos_embed.reshape(S, D).astype(jnp.float32),
        has_held_embed.reshape(D).astype(jnp.float32),
        type_embed.reshape(D).astype(jnp.float32),
    )
    return out.reshape(b, S, D)


# speedup vs baseline: 14.6663x; 14.6663x over previous
import functools

import jax
import jax.numpy as jnp
from jax import lax
from jax.experimental import pallas as pl
from jax.experimental.pallas import tpu as pltpu
from jax.experimental.pallas import tpu_sc as plsc

B = 16384
D = 128
PREVIEW = 5
S = 7
NC = 2
NS = 16
L = 16
NW = NC * NS
BPW = B // NW
IW = 128
NIDX = BPW // IW
HB = BPW // 2
UNITS = 2 * S
_ROLE_IDS = (0, 1, 2, 2, 2, 2, 2)


@functools.partial(
    pl.kernel,
    out_type=jax.ShapeDtypeStruct((S, B, D), jnp.float32),
    mesh=plsc.VectorSubcoreMesh(core_axis_name="c", subcore_axis_name="s"),
    compiler_params=pltpu.CompilerParams(needs_layout_passes=False),
    scratch_types=dict(
        cur_v=pltpu.VMEM((BPW,), jnp.int32),
        hold_v=pltpu.VMEM((BPW,), jnp.int32),
        hh_v=pltpu.VMEM((BPW,), jnp.float32),
        pv_v=pltpu.VMEM((PREVIEW, BPW), jnp.int32),
        pe_v=pltpu.VMEM((8, D), jnp.float32),
        role_v=pltpu.VMEM((3, D), jnp.float32),
        slot_v=pltpu.VMEM((S, D), jnp.float32),
        type_v=pltpu.VMEM((D,), jnp.float32),
        hhe_v=pltpu.VMEM((D,), jnp.float32),
        tbl_v=pltpu.VMEM((56, D), jnp.float32),
        shared_tbl=pltpu.VMEM_SHARED((56, D), jnp.float32),
        fidx=pltpu.VMEM((S * NIDX, IW), jnp.int32),
        rows0=pltpu.VMEM((HB, D), jnp.float32),
        rows1=pltpu.VMEM((HB, D), jnp.float32),
        gsem0=pltpu.SemaphoreType.DMA,
        gsem1=pltpu.SemaphoreType.DMA,
        osem0=pltpu.SemaphoreType.DMA,
        osem1=pltpu.SemaphoreType.DMA,
    ),
)
def _pq_embed_sc(cur_hbm, hold_hbm, hh_hbm, pvt_hbm, pe_hbm, role_hbm,
                 slot_hbm, hhe_hbm, type_hbm, out_hbm, *, cur_v, hold_v,
                 hh_v, pv_v, pe_v, role_v, slot_v, type_v, hhe_v, tbl_v,
                 shared_tbl, fidx, rows0, rows1, gsem0, gsem1, osem0,
                 osem1):
    cid = lax.axis_index("c")
    sid = lax.axis_index("s")
    wid = sid * NC + cid
    base = wid * BPW

    pltpu.sync_copy(cur_hbm.at[pl.ds(base, BPW)], cur_v)
    pltpu.sync_copy(hold_hbm.at[pl.ds(base, BPW)], hold_v)
    pltpu.sync_copy(hh_hbm.at[pl.ds(base, BPW)], hh_v)
    pltpu.sync_copy(pvt_hbm.at[:, pl.ds(base, BPW)], pv_v)
    pltpu.sync_copy(hhe_hbm, hhe_v)

    @pl.when(sid == 0)
    def _build_table():
        pltpu.sync_copy(pe_hbm, pe_v)
        pltpu.sync_copy(role_hbm, role_v)
        pltpu.sync_copy(slot_hbm, slot_v)
        pltpu.sync_copy(type_hbm, type_v)
        for d in range(D // L):
            dsl = pl.ds(d * L, L)
            t16 = type_v[dsl]
            for s in range(S):
                bias = role_v[_ROLE_IDS[s], dsl] + slot_v[s, dsl] + t16
                for p in range(8):
                    tbl_v[s * 8 + p, dsl] = pe_v[p, dsl] + bias
        pltpu.sync_copy(tbl_v, shared_tbl)

    plsc.subcore_barrier()

    for g in range(BPW // L):
        r, c = g // (IW // L), L * (g % (IW // L))
        gsl = pl.ds(g * L, L)
        csl = pl.ds(c, L)
        fidx[r, csl] = cur_v[gsl]
        fidx[NIDX + r, csl] = hold_v[gsl] + 8
        for k in range(PREVIEW):
            fidx[(2 + k) * NIDX + r, csl] = pv_v[k, gsl] + 8 * (2 + k)

    hhe_g = [hhe_v[pl.ds(d * L, L)] for d in range(D // L)]

    def start_gathers(u, rows, sem):
        s, h = u // 2, u % 2
        return [
            pltpu.async_copy(
                shared_tbl.at[fidx.at[s * NIDX + 2 * h + j]],
                rows.at[pl.ds(j * IW, IW)], sem)
            for j in range(HB // IW)
        ]

    def apply_fix(rows, h):
        def grp(g, carry):
            for b2 in range(L):
                hidx = jnp.full((L,), b2, jnp.int32) + (h * HB + g * L)
                splat = plsc.load_gather(hh_v, [hidx])
                r = g * L + b2
                for d in range(D // L):
                    dsl = pl.ds(d * L, L)
                    plsc.addupdate(rows.at[r, dsl], splat * hhe_g[d])
            return carry
        lax.fori_loop(0, HB // L, grp, 0)

    def start_out(u, rows, sem):
        s, h = u // 2, u % 2
        return pltpu.async_copy(
            rows, out_hbm.at[s, pl.ds(base + h * HB, HB)], sem)

    bufs = ((rows0, gsem0, osem0), (rows1, gsem1, osem1))
    pending_g = [start_gathers(0, rows0, gsem0), None]
    pending_o = [None, None]
    for u in range(UNITS):
        p = u % 2
        rows, gsem, osem = bufs[p]
        if u + 1 < UNITS:
            nrows, ngsem, nosem = bufs[1 - p]
            if pending_o[1 - p] is not None:
                pending_o[1 - p].wait()
            pending_g[1 - p] = start_gathers(u + 1, nrows, ngsem)
        for g in pending_g[p]:
            g.wait()
        if u // 2 == 1:
            apply_fix(rows, u % 2)
        pending_o[p] = start_out(u, rows, osem)
    pending_o[0].wait()
    pending_o[1].wait()


def kernel(current, hold, has_held, preview, piece_embed, role_embed,
           slot_pos_embed, has_held_embed, type_embed):
    b = current.shape[0]
    out = _pq_embed_sc(
        current.astype(jnp.int32),
        hold.astype(jnp.int32),
        has_held.astype(jnp.float32),
        preview.astype(jnp.int32).T,
        piece_embed.astype(jnp.float32),
        role_embed.astype(jnp.float32),
        slot_pos_embed.reshape(S, D).astype(jnp.float32),
        has_held_embed.reshape(D).astype(jnp.float32),
        type_embed.reshape(D).astype(jnp.float32),
    )
    return jnp.transpose(out, (1, 0, 2))
